# baseline (speedup 1.0000x reference)
import jax
import jax.numpy as jnp
from jax.experimental import pallas as pl
from jax.experimental.pallas import tpu as pltpu

_WB = 512


def _body(xt_ref, xw_ref, a_ref, p_ref, out_ref, lhs_ref):
    w = pl.program_id(0)

    @pl.when(w == 0)
    def _init():
        xt = xt_ref[...]
        lhs_ref[...] = jnp.where(jnp.isneginf(xt), 0.0, xt).astype(jnp.bfloat16)

    acc = jnp.dot(
        lhs_ref[...],
        a_ref[...].astype(jnp.bfloat16),
        preferred_element_type=jnp.float32,
    )
    xw = xw_ref[...]
    mask = jnp.isneginf(xw)
    imp = jnp.where(mask, 0.0, xw)
    half = imp.shape[0] // 2
    imp0, imp1 = imp[:half], imp[half:]
    x10, x11 = acc[:half], acc[half:]
    g0 = (p_ref[0, 0] * imp0 + p_ref[0, 1] * imp1
          + p_ref[0, 2] * x10 + p_ref[0, 3] * x11 + p_ref[0, 4])
    g1 = (p_ref[1, 0] * imp0 + p_ref[1, 1] * imp1
          + p_ref[1, 2] * x10 + p_ref[1, 3] * x11 + p_ref[1, 4])
    gcn = jnp.concatenate([g0, g1], axis=0)
    out_ref[...] = jnp.where(mask, gcn, imp)


def kernel(x, supports, W, b):
    B, C, N, L = x.shape
    R = C * B * L
    a = supports[0]
    xt = jnp.transpose(x, (1, 0, 3, 2)).reshape(R, N)
    params = jnp.concatenate([W, b[:, None]], axis=1)

    out_t = pl.pallas_call(
        _body,
        grid=(N // _WB,),
        in_specs=[
            pl.BlockSpec((R, N), lambda w: (0, 0)),
            pl.BlockSpec((R, _WB), lambda w: (0, w)),
            pl.BlockSpec((N, _WB), lambda w: (0, w)),
            pl.BlockSpec(memory_space=pltpu.SMEM),
        ],
        out_specs=pl.BlockSpec((R, _WB), lambda w: (0, w)),
        out_shape=jax.ShapeDtypeStruct((R, N), jnp.float32),
        scratch_shapes=[pltpu.VMEM((R, N), jnp.bfloat16)],
    )(xt, xt, a, params)

    return out_t.reshape(C, B, L, N).transpose(1, 0, 3, 2)
